# baseline (speedup 1.0000x reference)
import jax
import jax.numpy as jnp
from jax import lax
from jax.experimental import pallas as pl
from jax.experimental.pallas import tpu as pltpu
from jax.experimental.pallas import tpu_sc as plsc

VOCAB = 1000000
DIM = 64
B = 4096
X_MAX = 100.0
ALPHA = 0.75

NC = 2
NS = 16
NW = NC * NS
BPW = B // NW
L = 16

SPLIT = 500096
GRID = SPLIT // 128

_MESH = plsc.VectorSubcoreMesh(core_axis_name="c", subcore_axis_name="s")


def _conv_body(wa_ref, wb_ref, ca_ref, cb_ref, wo_ref, co_ref):
    wo_ref[:, :] = jnp.swapaxes(
        jnp.concatenate([wa_ref[:, :], wb_ref[:, :]], axis=0), 0, 1)
    co_ref[:, :] = jnp.swapaxes(
        jnp.concatenate([ca_ref[:, :], cb_ref[:, :]], axis=0), 0, 1)


_conv = pl.pallas_call(
    _conv_body,
    grid=(GRID,),
    in_specs=[
        pl.BlockSpec((DIM, 128), lambda g: (0, g)),
        pl.BlockSpec((DIM, 128), lambda g: (0, g + GRID)),
        pl.BlockSpec((DIM, 128), lambda g: (0, g)),
        pl.BlockSpec((DIM, 128), lambda g: (0, g + GRID)),
    ],
    out_specs=[
        pl.BlockSpec((128, 2 * DIM), lambda g: (g, 0)),
        pl.BlockSpec((128, 2 * DIM), lambda g: (g, 0)),
    ],
    out_shape=[
        jax.ShapeDtypeStruct((SPLIT, 2 * DIM), jnp.float32),
        jax.ShapeDtypeStruct((SPLIT, 2 * DIM), jnp.float32),
    ],
)


def _sc_body(widx_hbm, cidx_hbm, wpk_hbm, cpk_hbm, wbias_hbm, cbias_hbm,
             dots_out, b_out,
             widx_v, cidx_v, hw_v, hc_v, wrows_v, crows_v,
             wb_v, cb_v, dots_v, b_v, sem, bsem):
    wid = lax.axis_index("s") * NC + lax.axis_index("c")
    base = wid * BPW

    pltpu.sync_copy(widx_hbm.at[pl.ds(base, BPW)], widx_v)
    pltpu.sync_copy(cidx_hbm.at[pl.ds(base, BPW)], cidx_v)

    def half(c, carry):
        sl = pl.ds(c * L, L)
        jw = widx_v[sl]
        jc = cidx_v[sl]
        hw_v[sl] = jnp.where(jw >= SPLIT, jw - SPLIT, jw)
        hc_v[sl] = jnp.where(jc >= SPLIT, jc - SPLIT, jc)
        return carry

    lax.fori_loop(0, BPW // L, half, 0)

    ce = pltpu.async_copy(wpk_hbm.at[hw_v], wrows_v, sem)
    cf = pltpu.async_copy(cpk_hbm.at[hc_v], crows_v, sem)
    wbc = pltpu.async_copy(wbias_hbm.at[widx_v], wb_v, bsem)
    cbc = pltpu.async_copy(cbias_hbm.at[cidx_v], cb_v, bsem)
    ce.wait()
    cf.wait()
    wbc.wait()
    cbc.wait()

    def chunk(c, carry):
        sl = pl.ds(c * L, L)
        rows = lax.iota(jnp.int32, L) + c * L
        wbase = jnp.where(widx_v[sl] >= SPLIT, DIM, 0)
        cbase = jnp.where(cidx_v[sl] >= SPLIT, DIM, 0)

        def dstep(d, acc):
            vw = plsc.load_gather(wrows_v, [rows, wbase + d])
            vc = plsc.load_gather(crows_v, [rows, cbase + d])
            return acc + vw * vc

        dots_v[sl] = lax.fori_loop(0, DIM, dstep,
                                   jnp.zeros((L,), jnp.float32))
        b_v[sl] = wb_v[sl] + cb_v[sl]
        return carry

    lax.fori_loop(0, BPW // L, chunk, 0)

    pltpu.sync_copy(dots_v, dots_out.at[pl.ds(base, BPW)])
    pltpu.sync_copy(b_v, b_out.at[pl.ds(base, BPW)])


_sc_gather = pl.kernel(
    _sc_body,
    out_type=[
        jax.ShapeDtypeStruct((B,), jnp.float32),
        jax.ShapeDtypeStruct((B,), jnp.float32),
    ],
    mesh=_MESH,
    scratch_types=[
        pltpu.VMEM((BPW,), jnp.int32),
        pltpu.VMEM((BPW,), jnp.int32),
        pltpu.VMEM((BPW,), jnp.int32),
        pltpu.VMEM((BPW,), jnp.int32),
        pltpu.VMEM((BPW, 2 * DIM), jnp.float32),
        pltpu.VMEM((BPW, 2 * DIM), jnp.float32),
        pltpu.VMEM((BPW,), jnp.float32),
        pltpu.VMEM((BPW,), jnp.float32),
        pltpu.VMEM((BPW,), jnp.float32),
        pltpu.VMEM((BPW,), jnp.float32),
        pltpu.SemaphoreType.DMA,
        pltpu.SemaphoreType.DMA,
    ],
    compiler_params=pltpu.CompilerParams(needs_layout_passes=False),
)

_R = 32


def _tc_loss_body(dots_ref, b_ref, cooc_ref, out_ref):
    dots = dots_ref[:, :]
    b = b_ref[:, :]
    cc = cooc_ref[:, :]
    e = jnp.minimum(jnp.power(cc * (1.0 / X_MAX), ALPHA), 1.0)
    a = dots - jnp.log(cc + 1.0)
    s1 = jnp.sum(e * a * a)
    s2 = jnp.sum(e * a)
    s3 = jnp.sum(b)
    s4 = jnp.sum(b * b)
    s5 = jnp.sum(e)
    loss = s1 / B + (2.0 * s2 * s3 + s5 * s4) / (B * B)
    out_ref[:, :] = jnp.reshape(loss, (1, 1))


_tc_loss = pl.pallas_call(
    _tc_loss_body,
    out_shape=jax.ShapeDtypeStruct((1, 1), jnp.float32),
)


def kernel(word_input, context_input, coocurrence_count, word_emb_table,
           word_bias_table, context_emb_table, context_bias_table):
    wt = word_emb_table.T
    ct = context_emb_table.T
    wpk, cpk = _conv(wt, wt, ct, ct)
    dots, b = _sc_gather(
        word_input, context_input, wpk, cpk,
        word_bias_table[:, 0], context_bias_table[:, 0])
    loss = _tc_loss(dots.reshape(_R, B // _R), b.reshape(_R, B // _R),
                    coocurrence_count.reshape(_R, B // _R))
    return loss.reshape(())


# speedup vs baseline: 2.3605x; 2.3605x over previous
import jax
import jax.numpy as jnp
from jax import lax
from jax.experimental import pallas as pl
from jax.experimental.pallas import tpu as pltpu
from jax.experimental.pallas import tpu_sc as plsc

VOCAB = 1000000
DIM = 64
B = 4096
X_MAX = 100.0
ALPHA = 0.75

NC = 2
NS = 16
NW = NC * NS
BPW = B // NW
L = 16

_MESH = plsc.VectorSubcoreMesh(core_axis_name="c", subcore_axis_name="s")


def _sc_word_body(widx_hbm, wtab_hbm, wbias_hbm, rows_out, wb_out,
                  widx_v, rows_v, wb_v, sem, bsem):
    wid = lax.axis_index("s") * NC + lax.axis_index("c")
    base = wid * BPW
    pltpu.sync_copy(widx_hbm.at[pl.ds(base, BPW)], widx_v)
    wbc = pltpu.async_copy(wbias_hbm.at[widx_v], wb_v, bsem)

    def issue(g, carry):
        vw = widx_v[pl.ds(g * L, L)]
        for k in range(L):
            i = g * L + k
            pltpu.async_copy(wtab_hbm.at[pl.ds(vw[k], 1), :],
                             rows_v.at[pl.ds(i, 1), :], sem)
        return carry

    lax.fori_loop(0, BPW // L, issue, 0)
    pltpu.make_async_copy(wtab_hbm.at[pl.ds(0, BPW), :], rows_v, sem).wait()
    wbc.wait()
    pltpu.sync_copy(rows_v, rows_out.at[pl.ds(base, BPW)])
    pltpu.sync_copy(wb_v, wb_out.at[pl.ds(base, BPW)])


_sc_word = pl.kernel(
    _sc_word_body,
    out_type=[
        jax.ShapeDtypeStruct((B, DIM), jnp.float32),
        jax.ShapeDtypeStruct((B,), jnp.float32),
    ],
    mesh=_MESH,
    scratch_types=[
        pltpu.VMEM((BPW,), jnp.int32),
        pltpu.VMEM((BPW, DIM), jnp.float32),
        pltpu.VMEM((BPW,), jnp.float32),
        pltpu.SemaphoreType.DMA,
        pltpu.SemaphoreType.DMA,
    ],
)


def _sc_ctx_body(cidx_hbm, ctab_hbm, cbias_hbm, rows_out, cb_out,
                 cidx_v, rows_v, cb_v, sem, bsem):
    wid = lax.axis_index("s") * NC + lax.axis_index("c")
    base = wid * BPW
    pltpu.sync_copy(cidx_hbm.at[pl.ds(base, BPW)], cidx_v)
    cbc = pltpu.async_copy(cbias_hbm.at[cidx_v], cb_v, bsem)
    pltpu.async_copy(ctab_hbm.at[cidx_v], rows_v, sem).wait()
    cbc.wait()
    pltpu.sync_copy(rows_v, rows_out.at[pl.ds(base, BPW)])
    pltpu.sync_copy(cb_v, cb_out.at[pl.ds(base, BPW)])


_sc_ctx = pl.kernel(
    _sc_ctx_body,
    out_type=[
        jax.ShapeDtypeStruct((B, DIM), jnp.float32),
        jax.ShapeDtypeStruct((B,), jnp.float32),
    ],
    mesh=_MESH,
    scratch_types=[
        pltpu.VMEM((BPW,), jnp.int32),
        pltpu.VMEM((BPW, DIM), jnp.float32),
        pltpu.VMEM((BPW,), jnp.float32),
        pltpu.SemaphoreType.DMA,
        pltpu.SemaphoreType.DMA,
    ],
    compiler_params=pltpu.CompilerParams(use_tc_tiling_on_sc=False),
)

_R = 32


def _tc_loss_body(wrows_ref, crows_ref, wb_ref, cb_ref, cooc_ref, out_ref):
    dots = jnp.sum(wrows_ref[:, :] * crows_ref[:, :], axis=1)
    dots = dots.reshape(_R, B // _R)
    b = wb_ref[:, :] + cb_ref[:, :]
    cc = cooc_ref[:, :]
    e = jnp.minimum(jnp.power(cc * (1.0 / X_MAX), ALPHA), 1.0)
    a = dots - jnp.log(cc + 1.0)
    s1 = jnp.sum(e * a * a)
    s2 = jnp.sum(e * a)
    s3 = jnp.sum(b)
    s4 = jnp.sum(b * b)
    s5 = jnp.sum(e)
    loss = s1 / B + (2.0 * s2 * s3 + s5 * s4) / (B * B)
    out_ref[:, :] = jnp.reshape(loss, (1, 1))


_tc_loss = pl.pallas_call(
    _tc_loss_body,
    out_shape=jax.ShapeDtypeStruct((1, 1), jnp.float32),
)


def kernel(word_input, context_input, coocurrence_count, word_emb_table,
           word_bias_table, context_emb_table, context_bias_table):
    crows, cb = _sc_ctx(context_input, context_emb_table,
                        context_bias_table[:, 0])
    wrows, wb = _sc_word(word_input, word_emb_table, word_bias_table[:, 0])
    loss = _tc_loss(wrows, crows, wb.reshape(_R, B // _R),
                    cb.reshape(_R, B // _R),
                    coocurrence_count.reshape(_R, B // _R))
    return loss.reshape(())
